# baseline (speedup 1.0000x reference)
import functools

import jax
import jax.numpy as jnp
from jax import lax
from jax.experimental import pallas as pl
from jax.experimental.pallas import tpu as pltpu
from jax.experimental.pallas import tpu_sc as plsc

_RATIO = 0.5
_LANES = 16


def _radix_select(rows3d, kk):
    nb = rows3d.shape[0]
    bits = lax.bitcast_convert_type(rows3d, jnp.int32)
    skey = jnp.where(bits >= 0, bits, bits ^ jnp.int32(0x7FFFFFFF))
    skey = jnp.where(bits == jnp.int32(-2**31), jnp.int32(0), skey)

    def count_ge(c):
        return jnp.sum((skey >= c).astype(jnp.int32), axis=(1, 2),
                       keepdims=True)

    zero = jnp.zeros((nb, 1, 1), jnp.int32)
    c0 = count_ge(zero)
    t0 = jnp.where(c0 >= kk, jnp.int32(0), jnp.int32(-2**31))

    def body(i, t):
        bit = jnp.int32(30) - i
        cand = t + (jnp.int32(1) << bit)
        cnt = count_ge(cand)
        return jnp.where(cnt >= kk, cand, t)

    t = lax.fori_loop(0, 31, body, t0)
    n_gt = jnp.sum((skey > t).astype(jnp.int32), axis=(1, 2), keepdims=True)
    fb = jnp.where(t >= 0, t, t ^ jnp.int32(0x7FFFFFFF))
    return lax.bitcast_convert_type(fb, jnp.float32), kk - n_gt


def _scores_body(x_ref, w1_ref, b1_ref, w2_ref, b2_ref, out_ref):
    xb = x_ref[...]
    h = lax.dot_general(xb, w1_ref[...], (((1,), (1,)), ((), ())),
                        preferred_element_type=jnp.float32)
    h = jnp.maximum(h + b1_ref[...], 0.0)
    s = lax.dot_general(h, w2_ref[...], (((1,), (1,)), ((), ())),
                        preferred_element_type=jnp.float32)
    out_ref[...] = s + b2_ref[0, 0]


def _select_body(k, s_ref, th_ref, neq_ref):
    rows3d = s_ref[...]
    nb = rows3d.shape[0]
    th, neq = _radix_select(rows3d, jnp.int32(k))
    th_ref[...] = jnp.broadcast_to(th, (nb, 1, 128))
    neq_ref[...] = jnp.broadcast_to(neq, (nb, 1, 128))


def _compute_scores_select(x2d, W1, b1, W2, b2, nb, k, rows_tile):
    n, d = x2d.shape
    h = W1.shape[0]
    w2p = jnp.pad(W2, ((0, 7), (0, 0)))
    grid = n // rows_tile
    sc8 = pl.pallas_call(
        _scores_body,
        grid=(grid,),
        in_specs=[
            pl.BlockSpec((rows_tile, d), lambda i: (i, 0)),
            pl.BlockSpec((h, d), lambda i: (0, 0)),
            pl.BlockSpec((1, h), lambda i: (0, 0)),
            pl.BlockSpec((8, h), lambda i: (0, 0)),
            pl.BlockSpec((1, 1), lambda i: (0, 0)),
        ],
        out_specs=pl.BlockSpec((rows_tile, 8), lambda i: (i, 0)),
        out_shape=jax.ShapeDtypeStruct((n, 8), jnp.float32),
    )(x2d, W1, b1.reshape(1, h), w2p, b2.reshape(1, 1))
    scores2d = sc8[:, 0].reshape(nb, n // nb)
    s = n // nb
    th, neq = pl.pallas_call(
        functools.partial(_select_body, k),
        in_specs=[pl.BlockSpec((nb, s // 128, 128), lambda: (0, 0, 0))],
        out_specs=[
            pl.BlockSpec((nb, 1, 128), lambda: (0, 0, 0)),
            pl.BlockSpec((nb, 1, 128), lambda: (0, 0, 0)),
        ],
        out_shape=[
            jax.ShapeDtypeStruct((nb, 1, 128), jnp.float32),
            jax.ShapeDtypeStruct((nb, 1, 128), jnp.int32),
        ],
    )(scores2d.reshape(nb, s // 128, 128))
    return scores2d, th.reshape(nb, 128), neq.reshape(nb, 128)


def _sc_slice(x2d, sc8, th, neq, B, S, D, K):
    L = _LANES
    CH = 16
    NC, NS = 2, 16
    NW = NC * NS
    rows_per_w = (B * K) // NW
    mesh = plsc.VectorSubcoreMesh(core_axis_name="c", subcore_axis_name="s")

    @functools.partial(
        pl.kernel,
        out_type=(
            jax.ShapeDtypeStruct((B * K, D), jnp.float32),
            jax.ShapeDtypeStruct((B, K), jnp.float32),
        ),
        mesh=mesh,
        compiler_params=pltpu.CompilerParams(needs_layout_passes=False),
        scratch_types=[
            pltpu.VMEM((S,), jnp.float32),
            pltpu.VMEM((L,), jnp.float32),
            pltpu.VMEM((L,), jnp.int32),
            pltpu.VMEM((K + L,), jnp.int32),
            pltpu.VMEM((K + L,), jnp.float32),
            pltpu.VMEM_SHARED((B, K), jnp.int32),
            pltpu.VMEM((rows_per_w,), jnp.int32),
            pltpu.VMEM((CH, D), jnp.float32),
            pltpu.VMEM((CH, D), jnp.float32),
            pltpu.SemaphoreType.DMA,
            pltpu.SemaphoreType.DMA,
        ],
    )
    def sc_kernel(x_hbm, s_hbm, th_hbm, neq_hbm, outx_hbm, outs_hbm,
                  row_v, th_v, neq_v, idx_v, scv, sh_idx, gi_v,
                  rows_a, rows_b, sem_g, sem_w):
        cid = lax.axis_index("c")
        sid = lax.axis_index("s")

        @pl.when(sid < B)
        def _():
            b = sid
            pltpu.sync_copy(s_hbm.at[b], row_v)
            pltpu.sync_copy(th_hbm.at[b, pl.ds(0, L)], th_v)
            pltpu.sync_copy(neq_hbm.at[b, pl.ds(0, L)], neq_v)
            th_vec = th_v[...]
            neq_vec = neq_v[...]
            base = b * S

            def body(i, carry):
                off, eqc = carry
                sv = row_v[pl.ds(i * L, L)]
                m_gt = sv > th_vec
                m_eq = sv == th_vec
                eqcum = plsc.cumsum(m_eq.astype(jnp.int32))
                rank = eqcum + (eqc - 1)
                m_keep = m_gt | (m_eq & (rank < neq_vec))
                iv = lax.iota(jnp.int32, L) + (base + i * L)
                plsc.store_compressed(idx_v.at[pl.ds(off, L)], iv, mask=m_keep)
                plsc.store_compressed(scv.at[pl.ds(off, L)], sv, mask=m_keep)
                nk = plsc.all_reduce_population_count(m_keep)[0]
                ne = plsc.all_reduce_population_count(m_eq)[0]
                return off + nk, eqc + ne

            lax.fori_loop(0, S // L, body, (jnp.int32(0), jnp.int32(0)))
            pltpu.sync_copy(idx_v.at[pl.ds(0, K)], sh_idx.at[b])

            @pl.when(cid == 0)
            def _():
                pltpu.sync_copy(scv.at[pl.ds(0, K)], outs_hbm.at[b])

        plsc.subcore_barrier()

        w = sid * NC + cid
        g0 = w * rows_per_w
        b2 = g0 // K
        k0 = g0 - b2 * K
        nch = rows_per_w // CH

        pltpu.sync_copy(sh_idx.at[b2, pl.ds(k0, rows_per_w)], gi_v)

        def _gather(c, buf):
            return pltpu.async_copy(
                x_hbm.at[gi_v.at[pl.ds(c * CH, CH)]], buf, sem_g)

        def _wb(c, buf):
            return pltpu.async_copy(
                buf, outx_hbm.at[pl.ds(g0 + c * CH, CH)], sem_w)

        def _drain_g(buf):
            pltpu.make_async_copy(x_hbm.at[pl.ds(0, CH)], buf, sem_g).wait()

        def _drain_w(buf):
            pltpu.make_async_copy(
                buf, outx_hbm.at[pl.ds(g0, CH)], sem_w).wait()

        _gather(0, rows_a)
        _gather(1, rows_b)
        _drain_g(rows_a)
        _wb(0, rows_a)

        def pbody(j2, _):
            c = 2 * j2
            _drain_w(rows_a)
            _gather(c + 2, rows_a)
            _drain_g(rows_b)
            _wb(c + 1, rows_b)
            _drain_w(rows_b)
            _gather(c + 3, rows_b)
            _drain_g(rows_a)
            _wb(c + 2, rows_a)
            return 0

        lax.fori_loop(0, (nch - 2) // 2, pbody, 0)
        _drain_g(rows_b)
        _wb(nch - 1, rows_b)
        _drain_w(rows_a)
        _drain_w(rows_b)

    return sc_kernel(x2d, sc8, th, neq)


def kernel(x, W1, b1, W2, b2):
    B, S, D = x.shape
    K = max(int(S * _RATIO), 1)
    x2d = x.reshape(B * S, D)
    sc8, th, neq = _compute_scores_select(
        x2d, W1, b1, W2, b2, B, K, rows_tile=2048)
    outx, outs = _sc_slice(x2d, sc8, th, neq, B, S, D, K)
    return outx.reshape(B, K, D), outs


# speedup vs baseline: 1.0094x; 1.0094x over previous
import functools

import jax
import jax.numpy as jnp
from jax import lax
from jax.experimental import pallas as pl
from jax.experimental.pallas import tpu as pltpu
from jax.experimental.pallas import tpu_sc as plsc

_RATIO = 0.5
_LANES = 16


def _radix_select(rows3d, kk):
    nb = rows3d.shape[0]
    bits = lax.bitcast_convert_type(rows3d, jnp.int32)
    skey = jnp.where(bits >= 0, bits, bits ^ jnp.int32(0x7FFFFFFF))
    skey = jnp.where(bits == jnp.int32(-2**31), jnp.int32(0), skey)

    def count_ge(c):
        return jnp.sum((skey >= c).astype(jnp.int32), axis=(1, 2),
                       keepdims=True)

    zero = jnp.zeros((nb, 1, 1), jnp.int32)
    c0 = count_ge(zero)
    t0 = jnp.where(c0 >= kk, jnp.int32(0), jnp.int32(-2**31))

    def body(i, t):
        bit = jnp.int32(30) - i
        cand = t + (jnp.int32(1) << bit)
        cnt = count_ge(cand)
        return jnp.where(cnt >= kk, cand, t)

    t = lax.fori_loop(0, 31, body, t0)
    n_gt = jnp.sum((skey > t).astype(jnp.int32), axis=(1, 2), keepdims=True)
    fb = jnp.where(t >= 0, t, t ^ jnp.int32(0x7FFFFFFF))
    return lax.bitcast_convert_type(fb, jnp.float32), kk - n_gt


def _scores_body(x_ref, w1_ref, b1_ref, w2_ref, b2_ref, out_ref):
    xb = x_ref[...]
    h = lax.dot_general(xb, w1_ref[...], (((1,), (1,)), ((), ())),
                        preferred_element_type=jnp.float32)
    h = jnp.maximum(h + b1_ref[...], 0.0)
    s = lax.dot_general(h, w2_ref[...], (((1,), (1,)), ((), ())),
                        preferred_element_type=jnp.float32)
    out_ref[...] = s + b2_ref[0, 0]


def _select_body(k, s_ref, th_ref, neq_ref):
    rows3d = s_ref[...]
    nb = rows3d.shape[0]
    th, neq = _radix_select(rows3d, jnp.int32(k))
    th_ref[...] = jnp.broadcast_to(th, (nb, 1, 128))
    neq_ref[...] = jnp.broadcast_to(neq, (nb, 1, 128))


def _compute_scores_select(x2d, W1, b1, W2, b2, nb, k, rows_tile):
    n, d = x2d.shape
    h = W1.shape[0]
    grid = n // rows_tile
    sc8 = pl.pallas_call(
        _scores_body,
        grid=(grid,),
        in_specs=[
            pl.BlockSpec((rows_tile, d), lambda i: (i, 0)),
            pl.BlockSpec((h, d), lambda i: (0, 0)),
            pl.BlockSpec((1, h), lambda i: (0, 0)),
            pl.BlockSpec((8, h), lambda i: (0, 0)),
            pl.BlockSpec((1, 1), lambda i: (0, 0)),
        ],
        out_specs=pl.BlockSpec((rows_tile, 8), lambda i: (i, 0)),
        out_shape=jax.ShapeDtypeStruct((n, 8), jnp.float32),
    )(x2d, W1, b1.reshape(1, h), W2, b2.reshape(1, 1))
    scores2d = sc8[:, 0].reshape(nb, n // nb)
    s = n // nb
    th, neq = pl.pallas_call(
        functools.partial(_select_body, k),
        in_specs=[pl.BlockSpec((nb, s // 128, 128), lambda: (0, 0, 0))],
        out_specs=[
            pl.BlockSpec((nb, 1, 128), lambda: (0, 0, 0)),
            pl.BlockSpec((nb, 1, 128), lambda: (0, 0, 0)),
        ],
        out_shape=[
            jax.ShapeDtypeStruct((nb, 1, 128), jnp.float32),
            jax.ShapeDtypeStruct((nb, 1, 128), jnp.int32),
        ],
    )(scores2d.reshape(nb, s // 128, 128))
    return scores2d, th.reshape(nb, 128), neq.reshape(nb, 128)


def _sc_slice(x2d, sc8, th, neq, B, S, D, K):
    L = _LANES
    CH = 16
    NC, NS = 2, 16
    NW = NC * NS
    rows_per_w = (B * K) // NW
    mesh = plsc.VectorSubcoreMesh(core_axis_name="c", subcore_axis_name="s")

    @functools.partial(
        pl.kernel,
        out_type=(
            jax.ShapeDtypeStruct((B * K, D), jnp.float32),
            jax.ShapeDtypeStruct((B, K), jnp.float32),
        ),
        mesh=mesh,
        compiler_params=pltpu.CompilerParams(needs_layout_passes=False),
        scratch_types=[
            pltpu.VMEM((S,), jnp.float32),
            pltpu.VMEM((L,), jnp.float32),
            pltpu.VMEM((L,), jnp.int32),
            pltpu.VMEM((K + L,), jnp.int32),
            pltpu.VMEM((K + L,), jnp.float32),
            pltpu.VMEM_SHARED((B, K), jnp.int32),
            pltpu.VMEM((rows_per_w,), jnp.int32),
            pltpu.VMEM((CH, D), jnp.float32),
            pltpu.VMEM((CH, D), jnp.float32),
            pltpu.SemaphoreType.DMA,
            pltpu.SemaphoreType.DMA,
        ],
    )
    def sc_kernel(x_hbm, s_hbm, th_hbm, neq_hbm, outx_hbm, outs_hbm,
                  row_v, th_v, neq_v, idx_v, scv, sh_idx, gi_v,
                  rows_a, rows_b, sem_g, sem_w):
        cid = lax.axis_index("c")
        sid = lax.axis_index("s")

        @pl.when(sid < B)
        def _():
            b = sid
            pltpu.sync_copy(s_hbm.at[b], row_v)
            pltpu.sync_copy(th_hbm.at[b, pl.ds(0, L)], th_v)
            pltpu.sync_copy(neq_hbm.at[b, pl.ds(0, L)], neq_v)
            th_vec = th_v[...]
            neq_vec = neq_v[...]
            base = b * S

            def body(i, carry):
                off, eqc = carry
                sv = row_v[pl.ds(i * L, L)]
                m_gt = sv > th_vec
                m_eq = sv == th_vec
                eqcum = plsc.cumsum(m_eq.astype(jnp.int32))
                rank = eqcum + (eqc - 1)
                m_keep = m_gt | (m_eq & (rank < neq_vec))
                iv = lax.iota(jnp.int32, L) + (base + i * L)
                plsc.store_compressed(idx_v.at[pl.ds(off, L)], iv, mask=m_keep)
                plsc.store_compressed(scv.at[pl.ds(off, L)], sv, mask=m_keep)
                nk = plsc.all_reduce_population_count(m_keep)[0]
                ne = plsc.all_reduce_population_count(m_eq)[0]
                return off + nk, eqc + ne

            lax.fori_loop(0, S // L, body, (jnp.int32(0), jnp.int32(0)))
            pltpu.sync_copy(idx_v.at[pl.ds(0, K)], sh_idx.at[b])

            @pl.when(cid == 0)
            def _():
                pltpu.sync_copy(scv.at[pl.ds(0, K)], outs_hbm.at[b])

        plsc.subcore_barrier()

        w = sid * NC + cid
        g0 = w * rows_per_w
        b2 = g0 // K
        k0 = g0 - b2 * K
        nch = rows_per_w // CH

        pltpu.sync_copy(sh_idx.at[b2, pl.ds(k0, rows_per_w)], gi_v)

        def _gather(c, buf):
            return pltpu.async_copy(
                x_hbm.at[gi_v.at[pl.ds(c * CH, CH)]], buf, sem_g)

        def _wb(c, buf):
            return pltpu.async_copy(
                buf, outx_hbm.at[pl.ds(g0 + c * CH, CH)], sem_w)

        def _drain_g(buf):
            pltpu.make_async_copy(x_hbm.at[pl.ds(0, CH)], buf, sem_g).wait()

        def _drain_w(buf):
            pltpu.make_async_copy(
                buf, outx_hbm.at[pl.ds(g0, CH)], sem_w).wait()

        _gather(0, rows_a)
        _gather(1, rows_b)
        _drain_g(rows_a)
        _wb(0, rows_a)

        def pbody(j2, _):
            c = 2 * j2
            _drain_w(rows_a)
            _gather(c + 2, rows_a)
            _drain_g(rows_b)
            _wb(c + 1, rows_b)
            _drain_w(rows_b)
            _gather(c + 3, rows_b)
            _drain_g(rows_a)
            _wb(c + 2, rows_a)
            return 0

        lax.fori_loop(0, (nch - 2) // 2, pbody, 0)
        _drain_g(rows_b)
        _wb(nch - 1, rows_b)
        _drain_w(rows_a)
        _drain_w(rows_b)

    return sc_kernel(x2d, sc8, th, neq)


def kernel(x, W1, b1, W2, b2):
    B, S, D = x.shape
    K = max(int(S * _RATIO), 1)
    x2d = x.reshape(B * S, D)
    sc8, th, neq = _compute_scores_select(
        x2d, W1, b1, W2, b2, B, K, rows_tile=2048)
    outx, outs = _sc_slice(x2d, sc8, th, neq, B, S, D, K)
    return outx.reshape(B, K, D), outs
